# baseline (device time: 332456 ns/iter reference)
import jax
import jax.numpy as jnp
from jax import lax
from jax.experimental import pallas as pl
from jax.experimental.pallas import tpu as pltpu

N_DEV = 4
N_SUB = 4


def kernel(x, w_mat, scale_x, scale_w):
    m, _ = x.shape
    _, n = w_mat.shape
    m_per = m // N_DEV
    half = n // 2
    sub_w = half // N_SUB

    s = (scale_x[0] * scale_w[0]).astype(jnp.float32).reshape(1, 1)
    x8 = x.astype(jnp.float8_e5m2)
    w8 = w_mat.astype(jnp.float8_e5m2)

    def col0(d, sub):
        return d * half + sub * sub_w

    def body(x_ref, w_ref, s_ref, out_ref, bufs, ostage,
             send_sems, recv_sems, osem):
        my = lax.axis_index("i")
        right = lax.rem(my + 1, N_DEV)
        left = lax.rem(my + N_DEV - 1, N_DEV)
        nbr = (right, left)

        barrier_sem = pltpu.get_barrier_semaphore()
        for t in (left, right):
            pl.semaphore_signal(barrier_sem, inc=1, device_id=(t,),
                                device_id_type=pl.DeviceIdType.MESH)
        pl.semaphore_wait(barrier_sem, 2)

        def send_chunk(d, h):
            off = (N_DEV - 1 - h) if d == 0 else (1 + h)
            return lax.rem(my + off, N_DEV)

        def recv_chunk(d, h):
            off = (N_DEV - 2 - h) if d == 0 else (2 + h)
            return lax.rem(my + off, N_DEV)

        def addend(d, sub, c):
            xc = x_ref[pl.ds(c * m_per, m_per), :]
            wc = w_ref[:, col0(d, sub):col0(d, sub) + sub_w]
            return jnp.dot(xc, wc, preferred_element_type=jnp.float32)

        def start_hop(d, sub, h):
            rd = pltpu.make_async_remote_copy(
                src_ref=bufs.at[d, sub, h % 2],
                dst_ref=bufs.at[d, sub, (h + 1) % 2],
                send_sem=send_sems.at[d, sub, h],
                recv_sem=recv_sems.at[d, sub, h],
                device_id=(nbr[d],),
                device_id_type=pl.DeviceIdType.MESH,
            )
            rd.start()
            return rd

        rdmas = {}
        for sub in range(N_SUB):
            for d in range(2):
                bufs[d, sub, 0] = addend(
                    d, sub, send_chunk(d, 0)).astype(jnp.bfloat16)
                rdmas[d, sub] = start_hop(d, sub, 0)

        ocps = {}
        for h in range(N_DEV - 1):
            r_slot = (h + 1) % 2
            for sub in range(N_SUB):
                for d in range(2):
                    a = addend(d, sub, recv_chunk(d, h))
                    rdmas[d, sub].wait()
                    if h < N_DEV - 2:
                        bufs[d, sub, r_slot] = (
                            bufs[d, sub, r_slot].astype(jnp.float32) + a
                        ).astype(jnp.bfloat16)
                        rdmas[d, sub] = start_hop(d, sub, h + 1)
                    else:
                        k = (sub * 2 + d) % 2
                        if k in ocps:
                            ocps[k].wait()
                        ostage[k] = (
                            bufs[d, sub, r_slot].astype(jnp.float32) + a
                        ) * s_ref[0, 0]
                        ocp = pltpu.make_async_copy(
                            ostage.at[k],
                            out_ref.at[:, pl.ds(col0(d, sub), sub_w)],
                            osem.at[k],
                        )
                        ocp.start()
                        ocps[k] = ocp
        for k in ocps:
            ocps[k].wait()

    return pl.pallas_call(
        body,
        out_shape=jax.ShapeDtypeStruct((m_per, n), jnp.float32),
        in_specs=[
            pl.BlockSpec(memory_space=pltpu.MemorySpace.VMEM),
            pl.BlockSpec(memory_space=pltpu.MemorySpace.VMEM),
            pl.BlockSpec(memory_space=pltpu.MemorySpace.SMEM),
        ],
        out_specs=pl.BlockSpec(memory_space=pl.ANY),
        scratch_shapes=[
            pltpu.VMEM((2, N_SUB, 2, m_per, sub_w), jnp.bfloat16),
            pltpu.VMEM((2, m_per, sub_w), jnp.float32),
            pltpu.SemaphoreType.DMA((2, N_SUB, N_DEV - 1)),
            pltpu.SemaphoreType.DMA((2, N_SUB, N_DEV - 1)),
            pltpu.SemaphoreType.DMA((2,)),
        ],
        compiler_params=pltpu.CompilerParams(
            collective_id=0,
            vmem_limit_bytes=64 * 1024 * 1024,
        ),
    )(x8, w8, s)


# device time: 310239 ns/iter; 1.0716x vs baseline; 1.0716x over previous
import jax
import jax.numpy as jnp
from jax import lax
from jax.experimental import pallas as pl
from jax.experimental.pallas import tpu as pltpu

N_DEV = 4
N_SUB = 4


def kernel(x, w_mat, scale_x, scale_w):
    m, _ = x.shape
    _, n = w_mat.shape
    m_per = m // N_DEV
    half = n // 2
    sub_w = half // N_SUB

    sx = scale_x.reshape(1, 1)
    sw = scale_w.reshape(1, 1)

    def col0(d, sub):
        return d * half + sub * sub_w

    KS = [(sub, d) for sub in range(N_SUB) for d in range(2)]

    def body(x_ref, w_ref, sx_ref, sw_ref, out_ref, bufs, wstage, xstage,
             ostage, send_sems, recv_sems, wsems, xsems, osems):
        my = lax.axis_index("i")
        right = lax.rem(my + 1, N_DEV)
        left = lax.rem(my + N_DEV - 1, N_DEV)
        nbr = (right, left)

        barrier_sem = pltpu.get_barrier_semaphore()
        for t in (left, right):
            pl.semaphore_signal(barrier_sem, inc=1, device_id=(t,),
                                device_id_type=pl.DeviceIdType.MESH)
        pl.semaphore_wait(barrier_sem, 2)

        def send_chunk(d, h):
            off = (N_DEV - 1 - h) if d == 0 else (1 + h)
            return lax.rem(my + off, N_DEV)

        def recv_chunk(d, h):
            off = (N_DEV - 2 - h) if d == 0 else (2 + h)
            return lax.rem(my + off, N_DEV)

        def xdma(d, c):
            return pltpu.make_async_copy(
                x_ref.at[pl.ds(c * m_per, m_per), :], xstage.at[d],
                xsems.at[d])

        def wdma(slot, sub, d):
            return pltpu.make_async_copy(
                w_ref.at[:, pl.ds(col0(d, sub), sub_w)], wstage.at[slot],
                wsems.at[slot])

        def start_hop(d, sub, h):
            rd = pltpu.make_async_remote_copy(
                src_ref=bufs.at[d, sub, h % 2],
                dst_ref=bufs.at[d, sub, (h + 1) % 2],
                send_sem=send_sems.at[d, sub, h],
                recv_sem=recv_sems.at[d, sub, h],
                device_id=(nbr[d],),
                device_id_type=pl.DeviceIdType.MESH,
            )
            rd.start()
            return rd

        rdmas = {}
        ocps = {}
        for ph in range(N_DEV):
            h = ph - 1
            cs = [send_chunk(dd, 0) if ph == 0 else recv_chunk(dd, h)
                  for dd in range(2)]
            xs = [xdma(dd, cs[dd]) for dd in range(2)]
            for xc in xs:
                xc.start()
            wd = {}
            for k in range(min(2, len(KS))):
                sub, d = KS[k]
                wd[k] = wdma(k % 2, sub, d)
                wd[k].start()
            for xc in xs:
                xc.wait()
            for k, (sub, d) in enumerate(KS):
                wd[k].wait()
                a = jnp.dot(
                    xstage[d].astype(jnp.bfloat16),
                    wstage[k % 2].astype(jnp.bfloat16),
                    preferred_element_type=jnp.float32,
                )
                if k + 2 < len(KS):
                    nsub, nd = KS[k + 2]
                    wd[k + 2] = wdma(k % 2, nsub, nd)
                    wd[k + 2].start()
                if ph == 0:
                    bufs[d, sub, 0] = a.astype(jnp.bfloat16)
                    rdmas[d, sub] = start_hop(d, sub, 0)
                else:
                    r_slot = (h + 1) % 2
                    rdmas[d, sub].wait()
                    if h < N_DEV - 2:
                        bufs[d, sub, r_slot] = (
                            bufs[d, sub, r_slot].astype(jnp.float32) + a
                        ).astype(jnp.bfloat16)
                        rdmas[d, sub] = start_hop(d, sub, h + 1)
                    else:
                        o = k % 2
                        if o in ocps:
                            ocps[o].wait()
                        ostage[o] = (
                            bufs[d, sub, r_slot].astype(jnp.float32) + a
                        ) * (sx_ref[0, 0] * sw_ref[0, 0])
                        ocp = pltpu.make_async_copy(
                            ostage.at[o],
                            out_ref.at[:, pl.ds(col0(d, sub), sub_w)],
                            osems.at[o],
                        )
                        ocp.start()
                        ocps[o] = ocp
        for o in ocps:
            ocps[o].wait()

    return pl.pallas_call(
        body,
        out_shape=jax.ShapeDtypeStruct((m_per, n), jnp.float32),
        in_specs=[
            pl.BlockSpec(memory_space=pl.ANY),
            pl.BlockSpec(memory_space=pl.ANY),
            pl.BlockSpec(memory_space=pltpu.MemorySpace.SMEM),
            pl.BlockSpec(memory_space=pltpu.MemorySpace.SMEM),
        ],
        out_specs=pl.BlockSpec(memory_space=pl.ANY),
        scratch_shapes=[
            pltpu.VMEM((2, N_SUB, 2, m_per, sub_w), jnp.bfloat16),
            pltpu.VMEM((2, m_per, sub_w), jnp.float32),
            pltpu.VMEM((2, m_per, sub_w), jnp.float32),
            pltpu.VMEM((2, m_per, sub_w), jnp.float32),
            pltpu.SemaphoreType.DMA((2, N_SUB, N_DEV - 1)),
            pltpu.SemaphoreType.DMA((2, N_SUB, N_DEV - 1)),
            pltpu.SemaphoreType.DMA((2,)),
            pltpu.SemaphoreType.DMA((2,)),
            pltpu.SemaphoreType.DMA((2,)),
        ],
        compiler_params=pltpu.CompilerParams(
            collective_id=0,
            vmem_limit_bytes=64 * 1024 * 1024,
        ),
    )(x, w_mat, sx, sw)


# device time: 306760 ns/iter; 1.0838x vs baseline; 1.0113x over previous
import jax
import jax.numpy as jnp
from jax import lax
from jax.experimental import pallas as pl
from jax.experimental.pallas import tpu as pltpu

N_DEV = 4
N_SUB = 8


def kernel(x, w_mat, scale_x, scale_w):
    m, k_loc = x.shape
    _, n = w_mat.shape
    m_per = m // N_DEV
    half = n // 2
    sub_w = half // N_SUB

    sx = scale_x.reshape(1, 1)
    sw = scale_w.reshape(1, 1)

    def col0(d, sub):
        return d * half + sub * sub_w

    KS = [(sub, d) for sub in range(N_SUB) for d in range(2)]

    def body(x_ref, w_ref, sx_ref, sw_ref, out_ref, bufs, wstage, xstage,
             ostage, send_sems, recv_sems, wsems, xsems, osems):
        my = lax.axis_index("i")
        right = lax.rem(my + 1, N_DEV)
        left = lax.rem(my + N_DEV - 1, N_DEV)
        nbr = (right, left)

        barrier_sem = pltpu.get_barrier_semaphore()
        for t in (left, right):
            pl.semaphore_signal(barrier_sem, inc=1, device_id=(t,),
                                device_id_type=pl.DeviceIdType.MESH)
        pl.semaphore_wait(barrier_sem, 2)

        def send_chunk(d, h):
            off = (N_DEV - 1 - h) if d == 0 else (1 + h)
            return lax.rem(my + off, N_DEV)

        def recv_chunk(d, h):
            off = (N_DEV - 2 - h) if d == 0 else (2 + h)
            return lax.rem(my + off, N_DEV)

        def xdma(d, c):
            return pltpu.make_async_copy(
                x_ref.at[pl.ds(c * m_per, m_per), :], xstage.at[d],
                xsems.at[d])

        def wdma(slot, sub, d):
            return pltpu.make_async_copy(
                w_ref.at[:, pl.ds(col0(d, sub), sub_w)], wstage.at[slot],
                wsems.at[slot])

        def start_hop(d, sub, h):
            rd = pltpu.make_async_remote_copy(
                src_ref=bufs.at[d, sub, h % 2],
                dst_ref=bufs.at[d, sub, (h + 1) % 2],
                send_sem=send_sems.at[d, sub, h],
                recv_sem=recv_sems.at[d, sub, h],
                device_id=(nbr[d],),
                device_id_type=pl.DeviceIdType.MESH,
            )
            rd.start()
            return rd

        rdmas = {}
        ocps = {}
        for ph in range(N_DEV):
            h = ph - 1
            cs = [send_chunk(dd, 0) if ph == 0 else recv_chunk(dd, h)
                  for dd in range(2)]
            xs = [xdma(dd, cs[dd]) for dd in range(2)]
            for xc in xs:
                xc.start()
            wd = {}
            for k in range(min(2, len(KS))):
                sub, d = KS[k]
                wd[k] = wdma(k % 2, sub, d)
                wd[k].start()
            for xc in xs:
                xc.wait()
            for k, (sub, d) in enumerate(KS):
                wd[k].wait()
                a = jnp.dot(
                    xstage[d].astype(jnp.bfloat16),
                    wstage[k % 2].astype(jnp.bfloat16),
                    preferred_element_type=jnp.float32,
                )
                if k + 2 < len(KS):
                    nsub, nd = KS[k + 2]
                    wd[k + 2] = wdma(k % 2, nsub, nd)
                    wd[k + 2].start()
                if ph == 0:
                    bufs[d, sub, 0] = a.astype(jnp.bfloat16)
                    rdmas[d, sub] = start_hop(d, sub, 0)
                else:
                    r_slot = (h + 1) % 2
                    rdmas[d, sub].wait()
                    if h < N_DEV - 2:
                        bufs[d, sub, r_slot] = (
                            bufs[d, sub, r_slot].astype(jnp.float32) + a
                        ).astype(jnp.bfloat16)
                        rdmas[d, sub] = start_hop(d, sub, h + 1)
                    else:
                        o = k % 2
                        if o in ocps:
                            ocps[o].wait()
                        ostage[o] = (
                            bufs[d, sub, r_slot].astype(jnp.float32) + a
                        ) * (sx_ref[0, 0] * sw_ref[0, 0])
                        ocp = pltpu.make_async_copy(
                            ostage.at[o],
                            out_ref.at[:, pl.ds(col0(d, sub), sub_w)],
                            osems.at[o],
                        )
                        ocp.start()
                        ocps[o] = ocp
        for o in ocps:
            ocps[o].wait()

    return pl.pallas_call(
        body,
        out_shape=jax.ShapeDtypeStruct((m_per, n), jnp.float32),
        in_specs=[
            pl.BlockSpec(memory_space=pl.ANY),
            pl.BlockSpec(memory_space=pl.ANY),
            pl.BlockSpec(memory_space=pltpu.MemorySpace.SMEM),
            pl.BlockSpec(memory_space=pltpu.MemorySpace.SMEM),
        ],
        out_specs=pl.BlockSpec(memory_space=pl.ANY),
        scratch_shapes=[
            pltpu.VMEM((2, N_SUB, 2, m_per, sub_w), jnp.bfloat16),
            pltpu.VMEM((2, m_per, sub_w), jnp.float32),
            pltpu.VMEM((2, m_per, k_loc), jnp.float32),
            pltpu.VMEM((2, m_per, sub_w), jnp.float32),
            pltpu.SemaphoreType.DMA((2, N_SUB, N_DEV - 1)),
            pltpu.SemaphoreType.DMA((2, N_SUB, N_DEV - 1)),
            pltpu.SemaphoreType.DMA((2,)),
            pltpu.SemaphoreType.DMA((2,)),
            pltpu.SemaphoreType.DMA((2,)),
        ],
        compiler_params=pltpu.CompilerParams(
            collective_id=0,
            vmem_limit_bytes=64 * 1024 * 1024,
        ),
    )(x, w_mat, sx, sw)
